# baseline (speedup 1.0000x reference)
import jax
import jax.numpy as jnp
from jax.experimental import pallas as pl
from jax.experimental.pallas import tpu as pltpu

_ROWS = 43008
_W = 224
_GRID = 6
_BR = _ROWS // _GRID
_BW = 128


def _noising_kernel(t_ref, betas_bar_ref, x_ref, noise_ref, out_ref, noise_out_ref):
    t = t_ref[0]
    beta = betas_bar_ref[t, 0]
    sa = jnp.sqrt(1.0 - beta)
    sb = jnp.sqrt(beta)
    n = noise_ref[...]
    out_ref[...] = sa * x_ref[...] + sb * n
    noise_out_ref[...] = n


def kernel(x, t, betas_bar, noise):
    t_arr = jnp.asarray(t, dtype=jnp.int32).reshape((1,))
    x2 = x.reshape(_ROWS, _W)
    n2 = noise.reshape(_ROWS, _W)
    blk = (_BR, _BW)
    noised, noise_out = pl.pallas_call(
        _noising_kernel,
        grid=(_GRID, 2),
        in_specs=[
            pl.BlockSpec(memory_space=pltpu.SMEM),
            pl.BlockSpec(memory_space=pltpu.SMEM),
            pl.BlockSpec(blk, lambda i, j: (i, j)),
            pl.BlockSpec(blk, lambda i, j: (i, j)),
        ],
        out_specs=[
            pl.BlockSpec(blk, lambda i, j: (i, j)),
            pl.BlockSpec(blk, lambda i, j: (i, j)),
        ],
        out_shape=[
            jax.ShapeDtypeStruct((_ROWS, _W), x.dtype),
            jax.ShapeDtypeStruct((_ROWS, _W), x.dtype),
        ],
    )(t_arr, betas_bar, x2, n2)
    return noised.reshape(x.shape), noise_out.reshape(x.shape)


# speedup vs baseline: 1.0399x; 1.0399x over previous
import jax
import jax.numpy as jnp
from jax.experimental import pallas as pl
from jax.experimental.pallas import tpu as pltpu

_ROWS = 43008
_W = 224
_GRID = 4
_BR = _ROWS // _GRID
_BW = 128


def _noising_kernel(t_ref, betas_bar_ref, x_ref, noise_ref, out_ref, noise_out_ref):
    t = t_ref[0]
    beta = betas_bar_ref[t, 0]
    sa = jnp.sqrt(1.0 - beta)
    sb = jnp.sqrt(beta)
    n = noise_ref[...]
    out_ref[...] = sa * x_ref[...] + sb * n
    noise_out_ref[...] = n


def kernel(x, t, betas_bar, noise):
    t_arr = jnp.asarray(t, dtype=jnp.int32).reshape((1,))
    x2 = x.reshape(_ROWS, _W)
    n2 = noise.reshape(_ROWS, _W)
    blk = (_BR, _BW)
    noised, noise_out = pl.pallas_call(
        _noising_kernel,
        grid=(_GRID, 2),
        in_specs=[
            pl.BlockSpec(memory_space=pltpu.SMEM),
            pl.BlockSpec(memory_space=pltpu.SMEM),
            pl.BlockSpec(blk, lambda i, j: (i, j)),
            pl.BlockSpec(blk, lambda i, j: (i, j)),
        ],
        out_specs=[
            pl.BlockSpec(blk, lambda i, j: (i, j)),
            pl.BlockSpec(blk, lambda i, j: (i, j)),
        ],
        out_shape=[
            jax.ShapeDtypeStruct((_ROWS, _W), x.dtype),
            jax.ShapeDtypeStruct((_ROWS, _W), x.dtype),
        ],
    )(t_arr, betas_bar, x2, n2)
    return noised.reshape(x.shape), noise_out.reshape(x.shape)
